# Initial kernel scaffold; baseline (speedup 1.0000x reference)
import jax
import jax.numpy as jnp
from jax.experimental import pallas as pl


def kernel(idx, target, table):
    raise NotImplementedError("write your pallas kernel here")


# speedup vs baseline: 3.1732x; 3.1732x over previous
import functools

import jax
import jax.numpy as jnp
from jax import lax
from jax.experimental import pallas as pl
from jax.experimental.pallas import tpu as pltpu
from jax.experimental.pallas import tpu_sc as plsc

NC, NS, L = 2, 16, 16
NW = NC * NS

ROWS_PER_CHUNK = 4


def _sc_gather_loss(table, idx2, tgt2, n_rows, vocab):
    n_chunks = n_rows // ROWS_PER_CHUNK
    cpw = n_chunks // NW
    steps = vocab // L

    mesh = plsc.VectorSubcoreMesh(
        core_axis_name="c", subcore_axis_name="s",
        num_cores=NC, num_subcores=NS)

    @functools.partial(
        pl.kernel,
        out_type=(
            jax.ShapeDtypeStruct((n_rows, vocab), jnp.float32),
            jax.ShapeDtypeStruct((n_chunks, ROWS_PER_CHUNK + 1, L), jnp.float32),
        ),
        mesh=mesh,
        compiler_params=pltpu.CompilerParams(needs_layout_passes=False),
        scratch_types=[
            pltpu.VMEM((cpw, ROWS_PER_CHUNK), jnp.int32),
            pltpu.VMEM((cpw * ROWS_PER_CHUNK,), jnp.int32),
            pltpu.VMEM((ROWS_PER_CHUNK, vocab), jnp.float32),
            pltpu.VMEM((ROWS_PER_CHUNK, vocab), jnp.float32),
            pltpu.VMEM((ROWS_PER_CHUNK + 1, L), jnp.float32),
            pltpu.SemaphoreType.DMA,
            pltpu.SemaphoreType.DMA,
            pltpu.SemaphoreType.DMA,
            pltpu.SemaphoreType.DMA,
        ],
    )
    def body(table_hbm, idx_hbm, tgt_hbm, logits_hbm, parts_hbm,
             idx_v, tgt_v, buf0, buf1, parts_v,
             gsem0, gsem1, ssem0, ssem1):
        w = lax.axis_index("s") * NC + lax.axis_index("c")
        cbase = w * cpw

        pltpu.sync_copy(idx_hbm.at[pl.ds(cbase, cpw)], idx_v)
        pltpu.sync_copy(
            tgt_hbm.at[pl.ds(cbase * ROWS_PER_CHUNK, cpw * ROWS_PER_CHUNK)],
            tgt_v)

        lane = lax.iota(jnp.int32, L)
        mask4 = lane < ROWS_PER_CHUNK

        def start_gather(c, buf, sem):
            pltpu.async_copy(table_hbm.at[idx_v.at[c]], buf, sem)

        def wait_gather(c, buf, sem):
            pltpu.make_async_copy(table_hbm.at[idx_v.at[c]], buf, sem).wait()

        start_gather(0, buf0, gsem0)
        start_gather(1, buf1, gsem1)

        def do_chunk(c, buf, gsem, ssem):
            wait_gather(c, buf, gsem)

            def inner(i, accs):
                s = pl.ds(i * L, L)
                return tuple(a + jnp.exp(buf[j, s]) for j, a in enumerate(accs))

            zero = jnp.zeros((L,), jnp.float32)
            accs = lax.fori_loop(0, steps, inner, (zero,) * ROWS_PER_CHUNK)
            for j in range(ROWS_PER_CHUNK):
                parts_v[j, :] = accs[j]

            toff = c * ROWS_PER_CHUNK + jnp.where(mask4, lane, 0)
            tvec = plsc.load_gather(tgt_v, [toff], mask=mask4)
            vals = plsc.load_gather(buf, [lane, tvec], mask=mask4)
            parts_v[ROWS_PER_CHUNK, :] = jnp.where(mask4, vals, 0.0)
            pltpu.sync_copy(parts_v, parts_hbm.at[cbase + c])

            dst = logits_hbm.at[pl.ds((cbase + c) * ROWS_PER_CHUNK,
                                      ROWS_PER_CHUNK)]
            pltpu.async_copy(buf, dst, ssem).wait()

            @pl.when(c + 2 < cpw)
            def _():
                start_gather(c + 2, buf, gsem)

        def pair(p, carry):
            do_chunk(2 * p, buf0, gsem0, ssem0)
            do_chunk(2 * p + 1, buf1, gsem1, ssem1)
            return carry

        lax.fori_loop(0, cpw // 2, pair, 0)

    return body(table, idx2, tgt2)


def _tc_loss(sump, tgtv, n_rows):
    def body(sump_ref, tgtv_ref, out_ref):
        lse = jnp.log(jnp.sum(sump_ref[...], axis=1, keepdims=True))
        total = ((jnp.sum(lse) - jnp.sum(tgtv_ref[...]))
                 / jnp.float32(n_rows))
        out_ref[...] = jnp.broadcast_to(total, (1, 1))

    return pl.pallas_call(
        body,
        out_shape=jax.ShapeDtypeStruct((1, 1), jnp.float32),
    )(sump, tgtv)


def kernel(idx, target, table):
    b, s = idx.shape
    vocab = table.shape[1]
    n_rows = b * s

    idx2 = idx.reshape(-1).astype(jnp.int32).reshape(-1, ROWS_PER_CHUNK)
    tgt2 = target.reshape(-1).astype(jnp.int32)

    logits, parts = _sc_gather_loss(table, idx2, tgt2, n_rows, vocab)

    sump = parts[:, :ROWS_PER_CHUNK, :].reshape(n_rows, L)
    tgtv = parts[:, ROWS_PER_CHUNK, :ROWS_PER_CHUNK].reshape(n_rows, 1)
    loss = _tc_loss(sump, tgtv, n_rows)[0, 0]

    return (logits.reshape(b, s, vocab), loss)


# speedup vs baseline: 3.7813x; 1.1916x over previous
import functools

import jax
import jax.numpy as jnp
from jax import lax
from jax.experimental import pallas as pl
from jax.experimental.pallas import tpu as pltpu
from jax.experimental.pallas import tpu_sc as plsc

NC, NS, L = 2, 16, 16
NW = NC * NS

ROWS_PER_CHUNK = 4


def _sc_gather_loss(table, idx2, tgt2, n_rows, vocab):
    n_chunks = n_rows // ROWS_PER_CHUNK
    cpw = n_chunks // NW
    steps = vocab // L

    mesh = plsc.VectorSubcoreMesh(
        core_axis_name="c", subcore_axis_name="s",
        num_cores=NC, num_subcores=NS)

    @functools.partial(
        pl.kernel,
        out_type=(
            jax.ShapeDtypeStruct((n_rows, vocab), jnp.float32),
            jax.ShapeDtypeStruct((n_chunks, ROWS_PER_CHUNK + 1, L), jnp.float32),
        ),
        mesh=mesh,
        compiler_params=pltpu.CompilerParams(needs_layout_passes=False),
        scratch_types=[
            pltpu.VMEM((cpw, ROWS_PER_CHUNK), jnp.int32),
            pltpu.VMEM((cpw * ROWS_PER_CHUNK,), jnp.int32),
            pltpu.VMEM((ROWS_PER_CHUNK, vocab), jnp.float32),
            pltpu.VMEM((ROWS_PER_CHUNK, vocab), jnp.float32),
            pltpu.VMEM((ROWS_PER_CHUNK, vocab), jnp.float32),
            pltpu.VMEM((ROWS_PER_CHUNK + 1, L), jnp.float32),
            pltpu.SemaphoreType.DMA,
            pltpu.SemaphoreType.DMA,
            pltpu.SemaphoreType.DMA,
            pltpu.SemaphoreType.DMA,
            pltpu.SemaphoreType.DMA,
            pltpu.SemaphoreType.DMA,
        ],
    )
    def body(table_hbm, idx_hbm, tgt_hbm, logits_hbm, parts_hbm,
             idx_v, tgt_v, buf0, buf1, buf2, parts_v,
             gsem0, gsem1, gsem2, ssem0, ssem1, ssem2):
        w = lax.axis_index("s") * NC + lax.axis_index("c")
        cbase = w * cpw

        pltpu.sync_copy(idx_hbm.at[pl.ds(cbase, cpw)], idx_v)
        pltpu.sync_copy(
            tgt_hbm.at[pl.ds(cbase * ROWS_PER_CHUNK, cpw * ROWS_PER_CHUNK)],
            tgt_v)

        lane = lax.iota(jnp.int32, L)
        mask4 = lane < ROWS_PER_CHUNK
        bufs = (buf0, buf1, buf2)
        gsems = (gsem0, gsem1, gsem2)
        ssems = (ssem0, ssem1, ssem2)

        def start_gather(c, buf, sem):
            pltpu.async_copy(table_hbm.at[idx_v.at[c]], buf, sem)

        def wait_gather(c, buf, sem):
            pltpu.make_async_copy(table_hbm.at[idx_v.at[c]], buf, sem).wait()

        def logits_dst(c):
            return logits_hbm.at[pl.ds((cbase + c) * ROWS_PER_CHUNK,
                                       ROWS_PER_CHUNK)]

        def wait_scatter(c, buf, sem):
            pltpu.make_async_copy(buf, logits_dst(c), sem).wait()

        start_gather(0, buf0, gsem0)
        start_gather(1, buf1, gsem1)

        def do_chunk(c, k, prefetch):
            buf, gsem, ssem = bufs[k], gsems[k], ssems[k]
            wait_gather(c, buf, gsem)

            def inner(i, accs):
                s = pl.ds(i * L, L)
                return tuple(a + jnp.exp(buf[j, s]) for j, a in enumerate(accs))

            zero = jnp.zeros((L,), jnp.float32)
            accs = lax.fori_loop(0, steps, inner, (zero,) * ROWS_PER_CHUNK)
            for j in range(ROWS_PER_CHUNK):
                parts_v[j, :] = accs[j]

            toff = c * ROWS_PER_CHUNK + jnp.where(mask4, lane, 0)
            tvec = plsc.load_gather(tgt_v, [toff], mask=mask4)
            vals = plsc.load_gather(buf, [lane, tvec], mask=mask4)
            parts_v[ROWS_PER_CHUNK, :] = jnp.where(mask4, vals, 0.0)
            pltpu.sync_copy(parts_v, parts_hbm.at[cbase + c])

            pltpu.async_copy(buf, logits_dst(c), ssem)
            if prefetch:
                k2 = (k + 2) % 3
                @pl.when(c >= 1)
                def _():
                    wait_scatter(c - 1, bufs[k2], ssems[k2])
                start_gather(c + 2, bufs[k2], gsems[k2])

        def triple(p, carry):
            c = 3 * p
            do_chunk(c, 0, True)
            do_chunk(c + 1, 1, True)
            do_chunk(c + 2, 2, True)
            return carry

        n_triples = (cpw - 2) // 3
        lax.fori_loop(0, n_triples, triple, 0)
        for c in range(3 * n_triples, cpw):
            do_chunk(c, c % 3, False)
        for c in range(cpw - 3, cpw):
            wait_scatter(c, bufs[c % 3], ssems[c % 3])

    return body(table, idx2, tgt2)


def _tc_loss(parts, n_rows):
    def body(parts_ref, out_ref):
        p = parts_ref[...]
        lse = jnp.log(jnp.sum(p[:, :ROWS_PER_CHUNK, :], axis=2))
        total = ((jnp.sum(lse) - jnp.sum(p[:, ROWS_PER_CHUNK, :]))
                 / jnp.float32(n_rows))
        out_ref[...] = jnp.broadcast_to(total, (1, 1))

    return pl.pallas_call(
        body,
        out_shape=jax.ShapeDtypeStruct((1, 1), jnp.float32),
    )(parts)


def kernel(idx, target, table):
    b, s = idx.shape
    vocab = table.shape[1]
    n_rows = b * s

    idx2 = idx.reshape(-1).astype(jnp.int32).reshape(-1, ROWS_PER_CHUNK)
    tgt2 = target.reshape(-1).astype(jnp.int32)

    logits, parts = _sc_gather_loss(table, idx2, tgt2, n_rows, vocab)
    loss = _tc_loss(parts, n_rows)[0, 0]

    return (logits.reshape(b, s, vocab), loss)
